# baseline (speedup 1.0000x reference)
import functools

import jax
import jax.numpy as jnp
from jax import lax
from jax.experimental import pallas as pl
from jax.experimental.pallas import tpu as pltpu
from jax.experimental.pallas import tpu_sc as plsc

N = 10000
K = 32
C_IN = 128
C_ADD = 16
C_MID = 8
C_OUT = 128

NUM_WORKERS = 32
ROWS_PER_W = N * K // NUM_WORKERS
CHUNK = 400
SUB = 80
NCHUNKS = ROWS_PER_W // CHUNK
ROW_W = C_IN // 2

NB = 200
GRID = N // NB
GP = NB // 8


def _sc_gather(table_i32, idx_flat):
    mesh = plsc.VectorSubcoreMesh(core_axis_name="c", subcore_axis_name="s")

    @functools.partial(
        pl.kernel,
        out_type=jax.ShapeDtypeStruct((N * K, ROW_W), jnp.int32),
        mesh=mesh,
        scratch_types=[
            pltpu.VMEM((CHUNK,), jnp.int32),
            pltpu.VMEM((CHUNK, ROW_W), jnp.int32),
            pltpu.SemaphoreType.DMA,
        ],
        compiler_params=pltpu.CompilerParams(use_tc_tiling_on_sc=False),
    )
    def body(table_hbm, idx_hbm, out_hbm, idx_v, rows_v, sem):
        wid = lax.axis_index("s") * 2 + lax.axis_index("c")
        base = wid * ROWS_PER_W

        def chunk_body(t, carry):
            off = base + t * CHUNK
            pltpu.sync_copy(idx_hbm.at[pl.ds(off, CHUNK)], idx_v)
            cps = [
                pltpu.async_copy(
                    table_hbm.at[idx_v.at[pl.ds(j * SUB, SUB)]],
                    rows_v.at[pl.ds(j * SUB, SUB)],
                    sem,
                )
                for j in range(CHUNK // SUB)
            ]
            for c in cps:
                c.wait()
            pltpu.sync_copy(rows_v, out_hbm.at[pl.ds(off, CHUNK)])
            return carry

        lax.fori_loop(0, NCHUNKS, chunk_body, 0)

    return body(table_i32, idx_flat)


def _tc_body(g_ref, wnt_ref, add_ref, wg2_ref, wa2_ref, b_ref, out_ref,
             wf_scr, wfa_scr):
    rr = lax.broadcasted_iota(jnp.int32, (8 * K, 8 * C_MID), 0) // K
    qq = lax.broadcasted_iota(jnp.int32, (8 * K, 8 * C_MID), 1) % 8
    diag = rr == qq
    for g in range(GP):
        wblk = jnp.where(diag, wnt_ref[pl.ds(g * 8 * K, 8 * K)],
                         jnp.bfloat16(0))
        g8 = g_ref[pl.ds(g * 8 * K, 8 * K)]
        a8 = add_ref[pl.ds(g * 8 * K, 8 * K)]
        wfT = lax.dot_general(wblk, g8, (((0,), (0,)), ((), ())),
                              preferred_element_type=jnp.float32)
        wfaT = lax.dot_general(wblk, a8, (((0,), (0,)), ((), ())),
                               preferred_element_type=jnp.float32)
        wf_scr[:, pl.ds(g * 8, 8), :] = (
            wfT.astype(jnp.bfloat16).reshape(C_MID, 8, C_IN))
        wfa_scr[:, pl.ds(g * 8, 8), :] = (
            wfaT.astype(jnp.bfloat16).reshape(C_MID, 8, C_ADD))
    acc = jnp.zeros((NB, C_OUT), jnp.float32)
    for m in range(C_MID):
        acc = acc + lax.dot_general(
            wf_scr[m], wg2_ref[m],
            (((1,), (0,)), ((), ())), preferred_element_type=jnp.float32)
        acc = acc + lax.dot_general(
            wfa_scr[m], wa2_ref[m],
            (((1,), (0,)), ((), ())), preferred_element_type=jnp.float32)
    out_ref[...] = acc + b_ref[0]


def _tc_call(g_bf, wn_t, add_bf, wg2, wa2, bias2d):
    return pl.pallas_call(
        _tc_body,
        grid=(GRID,),
        in_specs=[
            pl.BlockSpec((NB * K, C_IN), lambda i: (i, 0)),
            pl.BlockSpec((NB * K, C_MID * 8), lambda i: (i, 0)),
            pl.BlockSpec((NB * K, C_ADD), lambda i: (i, 0)),
            pl.BlockSpec((C_MID, C_IN, C_OUT), lambda i: (0, 0, 0)),
            pl.BlockSpec((C_MID, C_ADD, C_OUT), lambda i: (0, 0, 0)),
            pl.BlockSpec((1, C_OUT), lambda i: (0, 0)),
        ],
        out_specs=pl.BlockSpec((NB, C_OUT), lambda i: (i, 0)),
        out_shape=jax.ShapeDtypeStruct((N, C_OUT), jnp.float32),
        scratch_shapes=[
            pltpu.VMEM((C_MID, NB, C_IN), jnp.bfloat16),
            pltpu.VMEM((C_MID, NB, C_ADD), jnp.bfloat16),
        ],
    )(g_bf, wn_t, add_bf, wg2, wa2, bias2d)


def kernel(input_features, neighbor_inds, weightnet, additional_features,
           linear_weight, linear_bias):
    B = input_features.shape[0]
    table_bf = input_features[0].astype(jnp.bfloat16)
    table_i32 = lax.bitcast_convert_type(
        table_bf.reshape(N, ROW_W, 2), jnp.int32)
    idx_flat = neighbor_inds[0].reshape(N * K)

    g_i32 = _sc_gather(table_i32, idx_flat)
    g_bf = lax.bitcast_convert_type(g_i32, jnp.bfloat16).reshape(N * K, C_IN)

    wn8 = weightnet[0].reshape(N * K, C_MID).astype(jnp.bfloat16)
    wn_t = jnp.broadcast_to(wn8[:, :, None],
                            (N * K, C_MID, 8)).reshape(N * K, C_MID * 8)
    add_bf = additional_features[0].reshape(N * K, C_ADD).astype(jnp.bfloat16)
    wr = linear_weight.reshape(C_OUT, C_IN + C_ADD, C_MID)
    wg2 = jnp.transpose(wr[:, :C_IN, :], (2, 1, 0)).astype(jnp.bfloat16)
    wa2 = jnp.transpose(wr[:, C_IN:, :], (2, 1, 0)).astype(jnp.bfloat16)

    out = _tc_call(g_bf, wn_t, add_bf, wg2, wa2, linear_bias.reshape(1, C_OUT))
    return out.reshape(B, N, C_OUT)
